# speedup vs baseline: 2.5353x; 2.5353x over previous
import jax
import jax.numpy as jnp
from jax.experimental import pallas as pl


def _copy_body(w_ref, o_ref):
    o_ref[...] = w_ref[...]


def kernel(emb, W):
    n_ctx, n_embd = W.shape
    seq_len = emb.shape[1]
    blk = 512
    grid = seq_len // blk
    out = pl.pallas_call(
        _copy_body,
        grid=(grid,),
        in_specs=[pl.BlockSpec((blk, n_embd), lambda i: (i, 0))],
        out_specs=pl.BlockSpec((blk, n_embd), lambda i: (i, 0)),
        out_shape=jax.ShapeDtypeStruct((seq_len, n_embd), jnp.float32),
    )(W)
    return out[None, :, :]
